# baseline (speedup 1.0000x reference)
import functools

import jax
import jax.numpy as jnp
from jax import lax
from jax.experimental import pallas as pl
from jax.experimental.pallas import tpu as pltpu
from jax.experimental.pallas import tpu_sc as plsc

N_NODES = 10000
DIM = 128
E_EDGES = 320000

NC = 2
NS = 16
NW = NC * NS

N_PAD = 10240
RPT = N_PAD // NS
K = 128
EPT = 10240
CHUNKS = EPT // K
HALF = CHUNKS // 2
E_PAD = NW * EPT

_mesh = plsc.VectorSubcoreMesh(core_axis_name="c", subcore_axis_name="s")


def _deg_body(src_hbm, zeros_hbm, ones_hbm, deg_out, src_v, ones_v, acc, dsem):
    c = lax.axis_index("c")
    s = lax.axis_index("s")
    w = c * NS + s
    zcp = pltpu.make_async_copy(zeros_hbm, acc.at[pl.ds(s * RPT, RPT)], dsem)
    zcp.start()
    pltpu.sync_copy(ones_hbm, ones_v)
    pltpu.sync_copy(src_hbm.at[w, pl.ds(0, CHUNKS // 2)], src_v)
    zcp.wait()
    plsc.subcore_barrier()

    def body(j, carry):
        pltpu.async_copy(ones_v, acc.at[src_v.at[j]], dsem, add=True)
        return carry

    def drain(j, carry):
        pltpu.make_async_copy(ones_v, acc.at[src_v.at[0]], dsem).wait()
        return carry

    for h in (0, 1):
        if h:
            pltpu.sync_copy(
                src_hbm.at[w, pl.ds(h * (CHUNKS // 2), CHUNKS // 2)], src_v)
        lax.fori_loop(0, CHUNKS // 2, body, 0)
        lax.fori_loop(0, CHUNKS // 2, drain, 0)
    plsc.subcore_barrier()
    pltpu.sync_copy(acc.at[pl.ds(s * RPT, RPT)],
                    deg_out.at[c, pl.ds(s * RPT, RPT)])


_deg_kernel = functools.partial(
    pl.kernel,
    out_type=jax.ShapeDtypeStruct((NC, N_PAD), jnp.float32),
    mesh=_mesh,
    scratch_types=[
        pltpu.VMEM((CHUNKS // 2, K), jnp.int32),
        pltpu.VMEM((K,), jnp.float32),
        pltpu.VMEM_SHARED((N_PAD,), jnp.float32),
        pltpu.SemaphoreType.DMA,
    ],
)(_deg_body)


def _agg_body(src_hbm, dst_hbm, xn_hbm, zeros_hbm, part_out,
              src_v, dst_v, rows_a, rows_b, acc, sem_a, sem_b):
    c = lax.axis_index("c")
    s = lax.axis_index("s")
    w = c * NS + s
    zcp = pltpu.make_async_copy(zeros_hbm, acc.at[pl.ds(s * RPT, RPT)], sem_a)
    zcp.start()
    pltpu.sync_copy(src_hbm.at[w, pl.ds(0, HALF)], src_v)
    pltpu.sync_copy(dst_hbm.at[w, pl.ds(0, HALF)], dst_v)
    zcp.wait()
    plsc.subcore_barrier()

    def process_half(h):
        if h:
            pltpu.sync_copy(src_hbm.at[w, pl.ds(h * HALF, HALF)], src_v)
            pltpu.sync_copy(dst_hbm.at[w, pl.ds(h * HALF, HALF)], dst_v)
        pltpu.async_copy(xn_hbm.at[src_v.at[0]], rows_a, sem_a)
        pltpu.async_copy(xn_hbm.at[src_v.at[1]], rows_b, sem_b)

        def body(t, carry):
            ja = 2 * t
            jb = 2 * t + 1
            pltpu.make_async_copy(xn_hbm.at[src_v.at[ja]], rows_a, sem_a).wait()
            pltpu.sync_copy(rows_a, acc.at[dst_v.at[ja]], add=True)
            pltpu.async_copy(xn_hbm.at[src_v.at[ja + 2]], rows_a, sem_a)
            pltpu.make_async_copy(xn_hbm.at[src_v.at[jb]], rows_b, sem_b).wait()
            pltpu.sync_copy(rows_b, acc.at[dst_v.at[jb]], add=True)
            pltpu.async_copy(xn_hbm.at[src_v.at[jb + 2]], rows_b, sem_b)
            return carry

        lax.fori_loop(0, HALF // 2 - 1, body, 0)
        ja = HALF - 2
        pltpu.make_async_copy(xn_hbm.at[src_v.at[ja]], rows_a, sem_a).wait()
        pltpu.sync_copy(rows_a, acc.at[dst_v.at[ja]], add=True)
        pltpu.make_async_copy(xn_hbm.at[src_v.at[ja + 1]], rows_b, sem_b).wait()
        pltpu.sync_copy(rows_b, acc.at[dst_v.at[ja + 1]], add=True)

    process_half(0)
    process_half(1)
    plsc.subcore_barrier()
    pltpu.sync_copy(acc.at[pl.ds(s * RPT, RPT)],
                    part_out.at[c, pl.ds(s * RPT, RPT)])


_agg_kernel = functools.partial(
    pl.kernel,
    out_type=jax.ShapeDtypeStruct((NC, N_PAD, DIM), jnp.float32),
    mesh=_mesh,
    scratch_types=[
        pltpu.VMEM((HALF, K), jnp.int32),
        pltpu.VMEM((HALF, K), jnp.int32),
        pltpu.VMEM((K, DIM), jnp.float32),
        pltpu.VMEM((K, DIM), jnp.float32),
        pltpu.VMEM_SHARED((N_PAD, DIM), jnp.float32),
        pltpu.SemaphoreType.DMA,
        pltpu.SemaphoreType.DMA,
    ],
)(_agg_body)


BN = 1000


def _mm1_body(x_ref, w1_ref, y_ref):
    y_ref[...] = jnp.dot(x_ref[...], w1_ref[...],
                         preferred_element_type=jnp.float32)


def _scale_body(deg_ref, y_ref, yn_ref):
    d = deg_ref[0] + deg_ref[1]
    norm = lax.rsqrt(jnp.maximum(d, 1.0))
    yn_ref[...] = y_ref[...] * norm


def _out_body(part_ref, deg_ref, b1_ref, w2_ref, b2_ref, out_ref):
    p = part_ref[0] + part_ref[1]
    d = deg_ref[0] + deg_ref[1]
    norm = lax.rsqrt(jnp.maximum(d, 1.0))
    h = jnp.maximum(p * norm + b1_ref[...], 0.0)
    out_ref[...] = (jnp.dot(h, w2_ref[...], preferred_element_type=jnp.float32)
                    + b2_ref[...])


def kernel(x, edge_index, W1, b1, W2, b2):
    src = edge_index[0]
    dst = edge_index[1]
    pad = E_PAD - E_EDGES
    pad_hi = (N_NODES + jnp.arange(pad, dtype=jnp.int32)
              % (N_PAD - N_NODES)).astype(jnp.int32)
    pad_lo = (jnp.arange(pad, dtype=jnp.int32) % N_NODES).astype(jnp.int32)
    src_deg = jnp.concatenate([src, pad_hi]).reshape(NW, CHUNKS, K)
    src_gat = jnp.concatenate([src, pad_lo]).reshape(NW, CHUNKS, K)
    dst_sc = jnp.concatenate([dst, pad_hi]).reshape(NW, CHUNKS, K)

    zeros_1d = jnp.zeros((RPT,), jnp.float32)
    ones_k = jnp.ones((K,), jnp.float32)
    zeros_2d = jnp.zeros((RPT, DIM), jnp.float32)

    y = pl.pallas_call(
        _mm1_body,
        grid=(N_NODES // BN,),
        in_specs=[
            pl.BlockSpec((BN, DIM), lambda i: (i, 0)),
            pl.BlockSpec((DIM, DIM), lambda i: (0, 0)),
        ],
        out_specs=pl.BlockSpec((BN, DIM), lambda i: (i, 0)),
        out_shape=jax.ShapeDtypeStruct((N_NODES, DIM), jnp.float32),
    )(x, W1)

    deg_parts = _deg_kernel(src_deg, zeros_1d, ones_k)
    deg3 = deg_parts.reshape(NC, N_PAD, 1)

    yn = pl.pallas_call(
        _scale_body,
        grid=(N_NODES // BN,),
        in_specs=[
            pl.BlockSpec((NC, BN, 1), lambda i: (0, i, 0)),
            pl.BlockSpec((BN, DIM), lambda i: (i, 0)),
        ],
        out_specs=pl.BlockSpec((BN, DIM), lambda i: (i, 0)),
        out_shape=jax.ShapeDtypeStruct((N_NODES, DIM), jnp.float32),
    )(deg3, y)

    parts = _agg_kernel(src_gat, dst_sc, yn, zeros_2d)

    out = pl.pallas_call(
        _out_body,
        grid=(N_NODES // BN,),
        in_specs=[
            pl.BlockSpec((NC, BN, DIM), lambda i: (0, i, 0)),
            pl.BlockSpec((NC, BN, 1), lambda i: (0, i, 0)),
            pl.BlockSpec((DIM,), lambda i: (0,)),
            pl.BlockSpec((DIM, DIM), lambda i: (0, 0)),
            pl.BlockSpec((DIM,), lambda i: (0,)),
        ],
        out_specs=pl.BlockSpec((BN, DIM), lambda i: (i, 0)),
        out_shape=jax.ShapeDtypeStruct((N_NODES, DIM), jnp.float32),
    )(parts, deg3, b1, W2, b2)
    return out


# speedup vs baseline: 1.0145x; 1.0145x over previous
import functools

import jax
import jax.numpy as jnp
from jax import lax
from jax.experimental import pallas as pl
from jax.experimental.pallas import tpu as pltpu
from jax.experimental.pallas import tpu_sc as plsc

N_NODES = 10000
DIM = 128
E_EDGES = 320000

NC = 2
NS = 16
NW = NC * NS

N_PAD = 10240
RPT = N_PAD // NS
K = 128
EPT = 10240
CHUNKS = EPT // K
HALF = CHUNKS // 2
E_PAD = NW * EPT

_mesh = plsc.VectorSubcoreMesh(core_axis_name="c", subcore_axis_name="s")


def _deg_body(src_hbm, zeros_hbm, ones_hbm, deg_out,
              src_v, src_v2, ones_v, acc, dsem):
    c = lax.axis_index("c")
    s = lax.axis_index("s")
    w = c * NS + s
    zcp = pltpu.make_async_copy(zeros_hbm, acc.at[pl.ds(s * RPT, RPT)], dsem)
    zcp.start()
    pltpu.sync_copy(ones_hbm, ones_v)
    pltpu.sync_copy(src_hbm.at[w, pl.ds(0, CHUNKS // 2)], src_v)
    zcp.wait()
    plsc.subcore_barrier()

    def body(idx_v):
        def _b(j, carry):
            pltpu.async_copy(ones_v, acc.at[idx_v.at[j]], dsem, add=True)
            return carry
        return _b

    def drain(j, carry):
        pltpu.make_async_copy(ones_v, acc.at[src_v.at[0]], dsem).wait()
        return carry

    lax.fori_loop(0, CHUNKS // 2, body(src_v), 0)
    pltpu.sync_copy(src_hbm.at[w, pl.ds(CHUNKS // 2, CHUNKS // 2)], src_v2)
    lax.fori_loop(0, CHUNKS // 2, body(src_v2), 0)
    lax.fori_loop(0, CHUNKS, drain, 0)
    plsc.subcore_barrier()
    pltpu.sync_copy(acc.at[pl.ds(s * RPT, RPT)],
                    deg_out.at[c, pl.ds(s * RPT, RPT)])


_deg_kernel = functools.partial(
    pl.kernel,
    out_type=jax.ShapeDtypeStruct((NC, N_PAD), jnp.float32),
    mesh=_mesh,
    scratch_types=[
        pltpu.VMEM((CHUNKS // 2, K), jnp.int32),
        pltpu.VMEM((CHUNKS // 2, K), jnp.int32),
        pltpu.VMEM((K,), jnp.float32),
        pltpu.VMEM_SHARED((N_PAD,), jnp.float32),
        pltpu.SemaphoreType.DMA,
    ],
)(_deg_body)


def _agg_body(src_hbm, dst_hbm, xn_hbm, zeros_hbm, part_out,
              src_v, dst_v, rows_a, rows_b, acc, sem_a, sem_b):
    c = lax.axis_index("c")
    s = lax.axis_index("s")
    w = c * NS + s
    zcp = pltpu.make_async_copy(zeros_hbm, acc.at[pl.ds(s * RPT, RPT)], sem_a)
    zcp.start()
    pltpu.sync_copy(src_hbm.at[w, pl.ds(0, HALF)], src_v)
    pltpu.sync_copy(dst_hbm.at[w, pl.ds(0, HALF)], dst_v)
    zcp.wait()
    plsc.subcore_barrier()

    def process_half(h):
        if h:
            pltpu.sync_copy(src_hbm.at[w, pl.ds(h * HALF, HALF)], src_v)
            pltpu.sync_copy(dst_hbm.at[w, pl.ds(h * HALF, HALF)], dst_v)
        pltpu.async_copy(xn_hbm.at[src_v.at[0]], rows_a, sem_a)
        pltpu.async_copy(xn_hbm.at[src_v.at[1]], rows_b, sem_b)

        def body(t, carry):
            ja = 2 * t
            jb = 2 * t + 1
            pltpu.make_async_copy(xn_hbm.at[src_v.at[ja]], rows_a, sem_a).wait()
            pltpu.sync_copy(rows_a, acc.at[dst_v.at[ja]], add=True)
            pltpu.async_copy(xn_hbm.at[src_v.at[ja + 2]], rows_a, sem_a)
            pltpu.make_async_copy(xn_hbm.at[src_v.at[jb]], rows_b, sem_b).wait()
            pltpu.sync_copy(rows_b, acc.at[dst_v.at[jb]], add=True)
            pltpu.async_copy(xn_hbm.at[src_v.at[jb + 2]], rows_b, sem_b)
            return carry

        lax.fori_loop(0, HALF // 2 - 1, body, 0)
        ja = HALF - 2
        pltpu.make_async_copy(xn_hbm.at[src_v.at[ja]], rows_a, sem_a).wait()
        pltpu.sync_copy(rows_a, acc.at[dst_v.at[ja]], add=True)
        pltpu.make_async_copy(xn_hbm.at[src_v.at[ja + 1]], rows_b, sem_b).wait()
        pltpu.sync_copy(rows_b, acc.at[dst_v.at[ja + 1]], add=True)

    process_half(0)
    process_half(1)
    plsc.subcore_barrier()
    pltpu.sync_copy(acc.at[pl.ds(s * RPT, RPT)],
                    part_out.at[c, pl.ds(s * RPT, RPT)])


_agg_kernel = functools.partial(
    pl.kernel,
    out_type=jax.ShapeDtypeStruct((NC, N_PAD, DIM), jnp.float32),
    mesh=_mesh,
    scratch_types=[
        pltpu.VMEM((HALF, K), jnp.int32),
        pltpu.VMEM((HALF, K), jnp.int32),
        pltpu.VMEM((K, DIM), jnp.float32),
        pltpu.VMEM((K, DIM), jnp.float32),
        pltpu.VMEM_SHARED((N_PAD, DIM), jnp.float32),
        pltpu.SemaphoreType.DMA,
        pltpu.SemaphoreType.DMA,
    ],
)(_agg_body)


BN = 1000


def _scale_body(deg_ref, x_ref, xn_ref):
    d = deg_ref[0] + deg_ref[1]
    norm = lax.rsqrt(jnp.maximum(d, 1.0))
    xn_ref[...] = x_ref[...] * norm


def _out_body(part_ref, deg_ref, w1_ref, b1_ref, w2_ref, b2_ref, out_ref):
    p = part_ref[0] + part_ref[1]
    d = deg_ref[0] + deg_ref[1]
    norm = lax.rsqrt(jnp.maximum(d, 1.0))
    h = p * norm
    h = jnp.dot(h, w1_ref[...], preferred_element_type=jnp.float32)
    h = jnp.maximum(h + b1_ref[...], 0.0)
    out_ref[...] = (jnp.dot(h, w2_ref[...], preferred_element_type=jnp.float32)
                    + b2_ref[...])


def kernel(x, edge_index, W1, b1, W2, b2):
    src = edge_index[0]
    dst = edge_index[1]
    pad = E_PAD - E_EDGES
    pad_hi = (N_NODES + jnp.arange(pad, dtype=jnp.int32)
              % (N_PAD - N_NODES)).astype(jnp.int32)
    pad_lo = (jnp.arange(pad, dtype=jnp.int32) % N_NODES).astype(jnp.int32)
    src_deg = jnp.concatenate([src, pad_hi]).reshape(NW, CHUNKS, K)
    src_gat = jnp.concatenate([src, pad_lo]).reshape(NW, CHUNKS, K)
    dst_sc = jnp.concatenate([dst, pad_hi]).reshape(NW, CHUNKS, K)

    zeros_1d = jnp.zeros((RPT,), jnp.float32)
    ones_k = jnp.ones((K,), jnp.float32)
    zeros_2d = jnp.zeros((RPT, DIM), jnp.float32)

    deg_parts = _deg_kernel(src_deg, zeros_1d, ones_k)
    deg3 = deg_parts.reshape(NC, N_PAD, 1)

    xn = pl.pallas_call(
        _scale_body,
        grid=(N_NODES // BN,),
        in_specs=[
            pl.BlockSpec((NC, BN, 1), lambda i: (0, i, 0)),
            pl.BlockSpec((BN, DIM), lambda i: (i, 0)),
        ],
        out_specs=pl.BlockSpec((BN, DIM), lambda i: (i, 0)),
        out_shape=jax.ShapeDtypeStruct((N_NODES, DIM), jnp.float32),
    )(deg3, x)

    parts = _agg_kernel(src_gat, dst_sc, xn, zeros_2d)

    out = pl.pallas_call(
        _out_body,
        grid=(N_NODES // BN,),
        in_specs=[
            pl.BlockSpec((NC, BN, DIM), lambda i: (0, i, 0)),
            pl.BlockSpec((NC, BN, 1), lambda i: (0, i, 0)),
            pl.BlockSpec((DIM, DIM), lambda i: (0, 0)),
            pl.BlockSpec((DIM,), lambda i: (0,)),
            pl.BlockSpec((DIM, DIM), lambda i: (0, 0)),
            pl.BlockSpec((DIM,), lambda i: (0,)),
        ],
        out_specs=pl.BlockSpec((BN, DIM), lambda i: (i, 0)),
        out_shape=jax.ShapeDtypeStruct((N_NODES, DIM), jnp.float32),
    )(parts, deg3, W1, b1, W2, b2)
    return out
